# Initial kernel scaffold; baseline (speedup 1.0000x reference)
import jax
import jax.numpy as jnp
from jax.experimental import pallas as pl


def kernel(x, memory_addresses, memory_values, W_q, W_out):
    raise NotImplementedError("write your pallas kernel here")


# speedup vs baseline: 41.2353x; 41.2353x over previous
import functools

import jax
import jax.numpy as jnp
from jax.experimental import pallas as pl

_TEMP = 0.25
_K = 8


def _block_kernel(x_ref, addr_ref, vals_ref, wq_ref, wout_ref, out_ref):
    xb = x_ref[0]
    addr = addr_ref[...]
    vals = vals_ref[0]
    wq = wq_ref[...]
    wout = wout_ref[...]

    q = jax.lax.dot_general(xb, wq, (((1,), (1,)), ((), ())),
                            preferred_element_type=jnp.float32)
    qn = jnp.sum(q * q, axis=-1, keepdims=True)
    q = q * jax.lax.rsqrt(jnp.maximum(qn, 1e-24))

    an = jnp.sum(addr * addr, axis=-1, keepdims=True)
    addr = addr * jax.lax.rsqrt(jnp.maximum(an, 1e-24))

    s = jax.lax.dot_general(q, addr, (((1,), (1,)), ((), ())),
                            preferred_element_type=jnp.float32)
    s = s * (1.0 / _TEMP)

    neg = jnp.float32(-jnp.inf)
    cur = s
    t = jnp.max(cur, axis=-1, keepdims=True)
    smax = t
    for _ in range(_K - 1):
        cur = jnp.where(cur >= t, neg, cur)
        t = jnp.max(cur, axis=-1, keepdims=True)

    sel = s >= t
    e = jnp.where(sel, jnp.exp(s - smax), 0.0)
    w = e / jnp.sum(e, axis=-1, keepdims=True)

    rv = jnp.dot(w, vals, preferred_element_type=jnp.float32)
    out_ref[0] = jax.lax.dot_general(rv, wout, (((1,), (1,)), ((), ())),
                                     preferred_element_type=jnp.float32)


@functools.partial(jax.jit, static_argnames=("interpret",))
def kernel(x, memory_addresses, memory_values, W_q, W_out, interpret=False):
    B, C, D = x.shape
    M, A = memory_addresses.shape
    CB = 1024
    grid = (B, C // CB)

    return pl.pallas_call(
        _block_kernel,
        grid=grid,
        in_specs=[
            pl.BlockSpec((1, CB, D), lambda b, c: (b, c, 0)),
            pl.BlockSpec((M, A), lambda b, c: (0, 0)),
            pl.BlockSpec((1, M, D), lambda b, c: (b, 0, 0)),
            pl.BlockSpec((A, D), lambda b, c: (0, 0)),
            pl.BlockSpec((D, D), lambda b, c: (0, 0)),
        ],
        out_specs=pl.BlockSpec((1, CB, D), lambda b, c: (b, c, 0)),
        out_shape=jax.ShapeDtypeStruct((B, C, D), jnp.float32),
        interpret=interpret,
    )(x, memory_addresses, memory_values, W_q, W_out)


# speedup vs baseline: 56.3560x; 1.3667x over previous
import functools

import jax
import jax.numpy as jnp
from jax.experimental import pallas as pl

_TEMP = 0.25
_K = 8


def _block_kernel(x_ref, addr_ref, vals_ref, wq_ref, wout_ref, out_ref):
    xb = x_ref[0]
    addr = addr_ref[...]
    vals = vals_ref[0]
    wq = wq_ref[...]
    wout = wout_ref[...]

    qT = jax.lax.dot_general(wq, xb, (((1,), (1,)), ((), ())),
                             preferred_element_type=jnp.float32)
    qn = jnp.sum(qT * qT, axis=0, keepdims=True)
    qT = qT * jax.lax.rsqrt(jnp.maximum(qn, 1e-24))

    an = jnp.sum(addr * addr, axis=-1, keepdims=True)
    addr = addr * jax.lax.rsqrt(jnp.maximum(an, 1e-24))

    sT = jnp.dot(addr, qT, preferred_element_type=jnp.float32)
    sT = sT * (1.0 / _TEMP)

    neg = jnp.float32(-jnp.inf)
    cur = sT
    t = jnp.max(cur, axis=0, keepdims=True)
    smax = t
    for _ in range(_K - 1):
        cur = jnp.where(cur >= t, neg, cur)
        t = jnp.max(cur, axis=0, keepdims=True)

    sel = sT >= t
    e = jnp.where(sel, jnp.exp(sT - smax), 0.0)
    w = e * (1.0 / jnp.sum(e, axis=0, keepdims=True))

    rv = jax.lax.dot_general(w, vals, (((0,), (0,)), ((), ())),
                             preferred_element_type=jnp.float32)
    out_ref[0] = jax.lax.dot_general(rv, wout, (((1,), (1,)), ((), ())),
                                     preferred_element_type=jnp.float32)


@functools.partial(jax.jit, static_argnames=("interpret",))
def kernel(x, memory_addresses, memory_values, W_q, W_out, interpret=False):
    B, C, D = x.shape
    M, A = memory_addresses.shape
    CB = 1024
    grid = (B, C // CB)

    return pl.pallas_call(
        _block_kernel,
        grid=grid,
        in_specs=[
            pl.BlockSpec((1, CB, D), lambda b, c: (b, c, 0)),
            pl.BlockSpec((M, A), lambda b, c: (0, 0)),
            pl.BlockSpec((1, M, D), lambda b, c: (b, 0, 0)),
            pl.BlockSpec((A, D), lambda b, c: (0, 0)),
            pl.BlockSpec((D, D), lambda b, c: (0, 0)),
        ],
        out_specs=pl.BlockSpec((1, CB, D), lambda b, c: (b, c, 0)),
        out_shape=jax.ShapeDtypeStruct((B, C, D), jnp.float32),
        interpret=interpret,
    )(x, memory_addresses, memory_values, W_q, W_out)


# speedup vs baseline: 68.7824x; 1.2205x over previous
import functools

import jax
import jax.numpy as jnp
from jax.experimental import pallas as pl

_TEMP = 0.25
_K = 8


def _block_kernel(x_ref, addr_ref, vals_ref, wq_ref, wout_ref, out_ref):
    xb = x_ref[0]
    addr = addr_ref[...]
    vals = vals_ref[0]
    wq = wq_ref[...]
    wout = wout_ref[...]

    qT = jax.lax.dot_general(wq, xb, (((1,), (1,)), ((), ())),
                             preferred_element_type=jnp.float32)
    qn = jnp.sum(qT * qT, axis=0, keepdims=True)
    qT = qT * jax.lax.rsqrt(jnp.maximum(qn, 1e-24))

    an = jnp.sum(addr * addr, axis=-1, keepdims=True)
    addr = addr * jax.lax.rsqrt(jnp.maximum(an, 1e-24))

    sT = jnp.dot(addr, qT, preferred_element_type=jnp.float32)
    sT = sT * (1.0 / _TEMP)

    neg = jnp.float32(-jnp.inf)
    cur = sT
    t = jnp.max(cur, axis=0, keepdims=True)
    smax = t
    for _ in range(_K - 1):
        cur = jnp.where(cur >= t, neg, cur)
        t = jnp.max(cur, axis=0, keepdims=True)

    sel = sT >= t
    e = jnp.where(sel, jnp.exp(sT - smax), 0.0)
    w = e * (1.0 / jnp.sum(e, axis=0, keepdims=True))

    rv = jax.lax.dot_general(w, vals, (((0,), (0,)), ((), ())),
                             preferred_element_type=jnp.float32)
    out_ref[0] = jax.lax.dot_general(rv, wout, (((1,), (1,)), ((), ())),
                                     preferred_element_type=jnp.float32)


@functools.partial(jax.jit, static_argnames=("interpret",))
def kernel(x, memory_addresses, memory_values, W_q, W_out, interpret=False):
    B, C, D = x.shape
    M, A = memory_addresses.shape
    CB = 2048
    grid = (B, C // CB)

    return pl.pallas_call(
        _block_kernel,
        grid=grid,
        in_specs=[
            pl.BlockSpec((1, CB, D), lambda b, c: (b, c, 0)),
            pl.BlockSpec((M, A), lambda b, c: (0, 0)),
            pl.BlockSpec((1, M, D), lambda b, c: (b, 0, 0)),
            pl.BlockSpec((A, D), lambda b, c: (0, 0)),
            pl.BlockSpec((D, D), lambda b, c: (0, 0)),
        ],
        out_specs=pl.BlockSpec((1, CB, D), lambda b, c: (b, c, 0)),
        out_shape=jax.ShapeDtypeStruct((B, C, D), jnp.float32),
        interpret=interpret,
    )(x, memory_addresses, memory_values, W_q, W_out)


# speedup vs baseline: 75.7374x; 1.1011x over previous
import functools

import jax
import jax.numpy as jnp
from jax.experimental import pallas as pl

_TEMP = 0.25
_K = 8


def _block_kernel(x_ref, addr_ref, vals_ref, wq_ref, wout_ref, out_ref):
    xb = x_ref[0]
    addr = addr_ref[...]
    vals = vals_ref[0]
    wq = wq_ref[...]
    wout = wout_ref[...]

    qT = jax.lax.dot_general(wq, xb, (((1,), (1,)), ((), ())),
                             preferred_element_type=jnp.float32)
    qn = jnp.sum(qT * qT, axis=0, keepdims=True)
    qT = qT * jax.lax.rsqrt(jnp.maximum(qn, 1e-24))

    an = jnp.sum(addr * addr, axis=-1, keepdims=True)
    addr = addr * jax.lax.rsqrt(jnp.maximum(an, 1e-24))

    sT = jnp.dot(addr, qT, preferred_element_type=jnp.float32)
    sT = sT * (1.0 / _TEMP)

    neg = jnp.float32(-jnp.inf)
    cur = sT
    t = jnp.max(cur, axis=0, keepdims=True)
    smax = t
    for _ in range(_K - 1):
        cur = jnp.where(cur >= t, neg, cur)
        t = jnp.max(cur, axis=0, keepdims=True)

    sel = sT >= t
    e = jnp.where(sel, jnp.exp(sT - smax), 0.0)
    w = e * (1.0 / jnp.sum(e, axis=0, keepdims=True))

    rv = jax.lax.dot_general(w, vals, (((0,), (0,)), ((), ())),
                             preferred_element_type=jnp.float32)
    out_ref[0] = jax.lax.dot_general(rv, wout, (((1,), (1,)), ((), ())),
                                     preferred_element_type=jnp.float32)


@functools.partial(jax.jit, static_argnames=("interpret",))
def kernel(x, memory_addresses, memory_values, W_q, W_out, interpret=False):
    B, C, D = x.shape
    M, A = memory_addresses.shape
    CB = 4096
    grid = (B, C // CB)

    return pl.pallas_call(
        _block_kernel,
        grid=grid,
        in_specs=[
            pl.BlockSpec((1, CB, D), lambda b, c: (b, c, 0)),
            pl.BlockSpec((M, A), lambda b, c: (0, 0)),
            pl.BlockSpec((1, M, D), lambda b, c: (b, 0, 0)),
            pl.BlockSpec((A, D), lambda b, c: (0, 0)),
            pl.BlockSpec((D, D), lambda b, c: (0, 0)),
        ],
        out_specs=pl.BlockSpec((1, CB, D), lambda b, c: (b, c, 0)),
        out_shape=jax.ShapeDtypeStruct((B, C, D), jnp.float32),
        interpret=interpret,
    )(x, memory_addresses, memory_values, W_q, W_out)


# speedup vs baseline: 77.8527x; 1.0279x over previous
import functools

import jax
import jax.numpy as jnp
from jax.experimental import pallas as pl

_TEMP = 0.25
_K = 8


def _block_kernel(x_ref, addr_ref, vals_ref, wq_ref, wout_ref, out_ref):
    xb = x_ref[0]
    addr = addr_ref[...]
    vals = vals_ref[0]
    wq = wq_ref[...]
    wout = wout_ref[...]

    qT = jax.lax.dot_general(wq, xb, (((1,), (1,)), ((), ())),
                             preferred_element_type=jnp.float32)
    qn = jnp.sum(qT * qT, axis=0, keepdims=True)
    qT = qT * jax.lax.rsqrt(jnp.maximum(qn, 1e-24))

    an = jnp.sum(addr * addr, axis=-1, keepdims=True)
    addr = addr * jax.lax.rsqrt(jnp.maximum(an, 1e-24))

    sT = jnp.dot(addr, qT, preferred_element_type=jnp.float32)
    sT = sT * (1.0 / _TEMP)

    neg = jnp.float32(-jnp.inf)
    cur = sT
    t = jnp.max(cur, axis=0, keepdims=True)
    smax = t
    for _ in range(_K - 1):
        cur = jnp.where(cur >= t, neg, cur)
        t = jnp.max(cur, axis=0, keepdims=True)

    sel = sT >= t
    e = jnp.where(sel, jnp.exp(sT - smax), 0.0)
    w = e * (1.0 / jnp.sum(e, axis=0, keepdims=True))

    rv = jax.lax.dot_general(w, vals, (((0,), (0,)), ((), ())),
                             preferred_element_type=jnp.float32)
    out_ref[0] = jax.lax.dot_general(rv, wout, (((1,), (1,)), ((), ())),
                                     preferred_element_type=jnp.float32)


@functools.partial(jax.jit, static_argnames=("interpret",))
def kernel(x, memory_addresses, memory_values, W_q, W_out, interpret=False):
    B, C, D = x.shape
    M, A = memory_addresses.shape
    CB = 8192
    grid = (B, C // CB)

    return pl.pallas_call(
        _block_kernel,
        grid=grid,
        in_specs=[
            pl.BlockSpec((1, CB, D), lambda b, c: (b, c, 0)),
            pl.BlockSpec((M, A), lambda b, c: (0, 0)),
            pl.BlockSpec((1, M, D), lambda b, c: (b, 0, 0)),
            pl.BlockSpec((A, D), lambda b, c: (0, 0)),
            pl.BlockSpec((D, D), lambda b, c: (0, 0)),
        ],
        out_specs=pl.BlockSpec((1, CB, D), lambda b, c: (b, c, 0)),
        out_shape=jax.ShapeDtypeStruct((B, C, D), jnp.float32),
        interpret=interpret,
    )(x, memory_addresses, memory_values, W_q, W_out)


# speedup vs baseline: 79.0307x; 1.0151x over previous
import functools

import jax
import jax.numpy as jnp
from jax.experimental import pallas as pl

_TEMP = 0.25
_K = 8


def _block_kernel(x_ref, addr_ref, vals_ref, wq_ref, wout_ref, out_ref):
    xb = x_ref[0]
    addr = addr_ref[...]
    vals = vals_ref[0]
    wq = wq_ref[...]
    wout = wout_ref[...]

    qT = jax.lax.dot_general(wq, xb, (((1,), (1,)), ((), ())),
                             preferred_element_type=jnp.float32)
    qn = jnp.sum(qT * qT, axis=0, keepdims=True)
    qT = qT * jax.lax.rsqrt(jnp.maximum(qn, 1e-24))

    an = jnp.sum(addr * addr, axis=-1, keepdims=True)
    addr = addr * jax.lax.rsqrt(jnp.maximum(an, 1e-24))

    sT = jnp.dot(addr, qT, preferred_element_type=jnp.float32)
    sT = sT * (1.0 / _TEMP)

    neg = jnp.float32(-jnp.inf)
    cur = sT
    t = jnp.max(cur, axis=0, keepdims=True)
    smax = t
    for _ in range(_K - 1):
        cur = jnp.where(cur >= t, neg, cur)
        t = jnp.max(cur, axis=0, keepdims=True)

    sel = sT >= t
    e = jnp.where(sel, jnp.exp(sT - smax), 0.0)
    w = e * (1.0 / jnp.sum(e, axis=0, keepdims=True))

    vw = jax.lax.dot_general(vals, wout, (((1,), (1,)), ((), ())),
                             preferred_element_type=jnp.float32)
    out_ref[0] = jax.lax.dot_general(w, vw, (((0,), (0,)), ((), ())),
                                     preferred_element_type=jnp.float32)


@functools.partial(jax.jit, static_argnames=("interpret",))
def kernel(x, memory_addresses, memory_values, W_q, W_out, interpret=False):
    B, C, D = x.shape
    M, A = memory_addresses.shape
    CB = 8192
    grid = (B, C // CB)

    return pl.pallas_call(
        _block_kernel,
        grid=grid,
        in_specs=[
            pl.BlockSpec((1, CB, D), lambda b, c: (b, c, 0)),
            pl.BlockSpec((M, A), lambda b, c: (0, 0)),
            pl.BlockSpec((1, M, D), lambda b, c: (b, 0, 0)),
            pl.BlockSpec((A, D), lambda b, c: (0, 0)),
            pl.BlockSpec((D, D), lambda b, c: (0, 0)),
        ],
        out_specs=pl.BlockSpec((1, CB, D), lambda b, c: (b, c, 0)),
        out_shape=jax.ShapeDtypeStruct((B, C, D), jnp.float32),
        interpret=interpret,
    )(x, memory_addresses, memory_values, W_q, W_out)
